# baseline (speedup 1.0000x reference)
import dataclasses
import functools

import jax
import jax.numpy as jnp
from jax import lax
from jax.experimental import pallas as pl
from jax.experimental.pallas import tpu as pltpu
from jax.experimental.pallas import tpu_sc as plsc

N_T = 10000
N_S = 2500
N_TP = 10240
N_SP = 2560
E1P = 327680
E2P = 40960
EPP = 12288
R1 = E1P // 128
R2 = E2P // 128
RP = EPP // 128

_mesh = plsc.VectorSubcoreMesh(core_axis_name="c", subcore_axis_name="s")


_cp = pltpu.CompilerParams()
if "needs_layout_passes" in pltpu.CompilerParams.__dataclass_fields__:
  _cp = dataclasses.replace(_cp, needs_layout_passes=False)


def _sc_hists(s1a, d1a, s1b, d1b, si2, di2, zeros128, iota128):
  @functools.partial(
      pl.kernel,
      out_type=(
          jax.ShapeDtypeStruct((2, 128, 128), jnp.float32),
          jax.ShapeDtypeStruct((2, 128, 128), jnp.float32),
          jax.ShapeDtypeStruct((128, 128), jnp.float32),
          jax.ShapeDtypeStruct((128, 128), jnp.float32),
      ),
      mesh=_mesh,
      scratch_types=[
          pltpu.VMEM_SHARED((128, 128), jnp.float32),
          pltpu.VMEM_SHARED((128, 128), jnp.float32),
          pltpu.VMEM_SHARED((128, 128), jnp.float32),
          pltpu.VMEM_SHARED((128, 128), jnp.float32),
          pltpu.VMEM((88, 128), jnp.int32),
          pltpu.VMEM((88, 128), jnp.int32),
          pltpu.VMEM((128, 128), jnp.float32),
          pltpu.VMEM((128, 128), jnp.float32),
          pltpu.VMEM((1, 128), jnp.int32),
      ],
      compiler_params=_cp,
  )
  def k(s1a_h, d1a_h, s1b_h, d1b_h, si2_h, di2_h, z128_h, iota_h, o_hs1,
        o_hd1, o_hs2, o_hd2, HsA, HsB, HsC, HsD, idx_s, idx_d, localA,
        localB, iota_v):
    ci = lax.axis_index("c")
    ti = lax.axis_index("s")
    z128 = pl.ds(0, 128)
    sl8 = pl.ds(ti * 8, 8)
    pltpu.sync_copy(z128_h.at[z128], localA)
    pltpu.sync_copy(z128_h.at[z128], localB)
    pltpu.sync_copy(z128_h.at[sl8], HsA.at[sl8])
    pltpu.sync_copy(z128_h.at[sl8], HsB.at[sl8])
    pltpu.sync_copy(z128_h.at[sl8], HsC.at[sl8])
    pltpu.sync_copy(z128_h.at[sl8], HsD.at[sl8])
    pltpu.sync_copy(iota_h, iota_v)
    ones = jnp.ones((16,), jnp.float32)

    def count(idx_ref, local_ref, nrows):
      @pl.loop(0, nrows)
      def _(j):
        @pl.loop(0, 128, step=16)
        def _(c):
          x = idx_ref[j, pl.ds(c, 16)]
          row = lax.shift_right_logical(x, 7)
          col = lax.bitwise_and(x, 127)
          plsc.addupdate_scatter(local_ref, [row, col], ones)

    @pl.when(ci == 0)
    def _():
      pltpu.sync_copy(s1a_h.at[ti], idx_s)
      pltpu.sync_copy(d1a_h.at[ti], idx_d)
      count(idx_s, localA, 88)
      count(idx_d, localB, 88)

    @pl.when(ci == 1)
    def _():
      pltpu.sync_copy(s1b_h.at[ti], idx_s.at[pl.ds(0, 72)])
      pltpu.sync_copy(d1b_h.at[ti], idx_d.at[pl.ds(0, 72)])
      count(idx_s, localA, 72)
      count(idx_d, localB, 72)

    plsc.subcore_barrier()
    pltpu.sync_copy(localA, HsA.at[iota_v.at[0]], add=True)
    pltpu.sync_copy(localB, HsB.at[iota_v.at[0]], add=True)

    @pl.when(ci == 1)
    def _():
      pltpu.sync_copy(z128_h.at[z128], localA)
      pltpu.sync_copy(z128_h.at[z128], localB)
      pltpu.sync_copy(si2_h.at[ti], idx_s.at[pl.ds(0, 20)])
      pltpu.sync_copy(di2_h.at[ti], idx_d.at[pl.ds(0, 20)])
      count(idx_s, localA, 20)
      count(idx_d, localB, 20)
      pltpu.sync_copy(localA, HsC.at[iota_v.at[0]], add=True)
      pltpu.sync_copy(localB, HsD.at[iota_v.at[0]], add=True)

    plsc.subcore_barrier()
    pltpu.sync_copy(HsA.at[sl8], o_hs1.at[ci, sl8])
    pltpu.sync_copy(HsB.at[sl8], o_hd1.at[ci, sl8])

    @pl.when(ci == 1)
    def _():
      pltpu.sync_copy(HsC.at[sl8], o_hs2.at[sl8])
      pltpu.sync_copy(HsD.at[sl8], o_hd2.at[sl8])

  return k(s1a, d1a, s1b, d1b, si2, di2, zeros128, iota128)


def _sc_agg(tab1, tab2, ptab, s1m, d1m, s1e, d1e, s2, d2, sp, dp, zeros128):
  @functools.partial(
      pl.kernel,
      out_type=(
          jax.ShapeDtypeStruct((2, N_TP, 128), jnp.float32),
          jax.ShapeDtypeStruct((N_TP, 128), jnp.float32),
          jax.ShapeDtypeStruct((N_TP, 128), jnp.float32),
      ),
      mesh=_mesh,
      scratch_types=[
          pltpu.VMEM_SHARED((N_TP, 128), jnp.float32),
          pltpu.VMEM((16, 128), jnp.int32),
          pltpu.VMEM((16, 128), jnp.int32),
          pltpu.VMEM((128, 128), jnp.float32),
          pltpu.VMEM((128, 128), jnp.float32),
          pltpu.SemaphoreType.DMA,
          pltpu.SemaphoreType.DMA,
          pltpu.SemaphoreType.DMA,
          pltpu.SemaphoreType.DMA,
      ],
  )
  def k(tab1_h, tab2_h, ptab_h, s1m_h, d1m_h, s1e_h, d1e_h, s2_h, d2_h, sp_h,
        dp_h, z128_h, o_p1, o_q2, o_ov, acc, idx_s, idx_d, rows0, rows1,
        gs0, gs1, ss0, ss1):
    ci = lax.axis_index("c")
    ti = lax.axis_index("s")
    sl = pl.ds(ti * 640, 640)

    def chunk(tab_h, s3_h, d3_h, widx, r0, n):
      pltpu.sync_copy(s3_h.at[widx, pl.ds(r0, n)], idx_s.at[pl.ds(0, n)])
      pltpu.sync_copy(d3_h.at[widx, pl.ds(r0, n)], idx_d.at[pl.ds(0, n)])

      @pl.loop(0, n, step=2)
      def _(j):
        g0 = pltpu.async_copy(tab_h.at[idx_s.at[j]], rows0, gs0)
        g1 = pltpu.async_copy(tab_h.at[idx_s.at[j + 1]], rows1, gs1)
        g0.wait()
        s0 = pltpu.async_copy(rows0, acc.at[idx_d.at[j]], ss0, add=True)
        g1.wait()
        s1 = pltpu.async_copy(rows1, acc.at[idx_d.at[j + 1]], ss1, add=True)
        s0.wait()
        s1.wait()

    pltpu.sync_copy(z128_h.at[sl], acc.at[sl])
    plsc.subcore_barrier()
    w = ci * 16 + ti

    @pl.loop(0, 72, step=8)
    def _(r):
      chunk(tab1_h, s1m_h, d1m_h, w, r, 8)

    @pl.when(ci == 0)
    def _():
      chunk(tab1_h, s1e_h, d1e_h, ti, 0, 16)

    plsc.subcore_barrier()
    pltpu.sync_copy(acc.at[sl], o_p1.at[ci, sl])
    plsc.subcore_barrier()

    pltpu.sync_copy(z128_h.at[sl], acc.at[sl])
    plsc.subcore_barrier()

    @pl.when(ci == 1)
    def _():
      chunk(tab2_h, s2_h, d2_h, ti, 0, 16)
      chunk(tab2_h, s2_h, d2_h, ti, 16, 4)
      plsc.subcore_barrier()
      pltpu.sync_copy(acc.at[sl], o_q2.at[sl])

    @pl.when(ci == 0)
    def _():
      chunk(ptab_h, sp_h, dp_h, ti, 0, 6)
      plsc.subcore_barrier()
      pltpu.sync_copy(acc.at[sl], o_ov.at[sl])

  return k(tab1, tab2, ptab, s1m, d1m, s1e, d1e, s2, d2, sp, dp, zeros128)


def _leaky(x):
  return jnp.where(x >= 0, x, 0.01 * x)


def _tc_tab1(feat, W1, hs1a, hs1b):
  def body(f_ref, w_ref, ha_ref, hb_ref, o_ref):
    h = jnp.dot(f_ref[...], w_ref[...], preferred_element_type=jnp.float32)
    o_ref[...] = h * lax.rsqrt(ha_ref[...] + hb_ref[...] + 1.0)

  return pl.pallas_call(
      body,
      grid=(5,),
      in_specs=[
          pl.BlockSpec((2000, 128), lambda i: (i, 0)),
          pl.BlockSpec((128, 128), lambda i: (0, 0)),
          pl.BlockSpec((2000, 1), lambda i: (i, 0)),
          pl.BlockSpec((2000, 1), lambda i: (i, 0)),
      ],
      out_specs=pl.BlockSpec((2000, 128), lambda i: (i, 0)),
      out_shape=jax.ShapeDtypeStruct((N_T, 128), jnp.float32),
  )(feat, W1, hs1a, hs1b)


def _tc_tab2(hs_new, Wp, bp, W2, hs2):
  def body(x_ref, wp_ref, bp_ref, w2_ref, h_ref, o_ref):
    fm = jnp.dot(x_ref[...], wp_ref[...],
                 preferred_element_type=jnp.float32) + bp_ref[...]
    h2 = jnp.dot(fm, w2_ref[...], preferred_element_type=jnp.float32)
    deg = h_ref[...]
    ns = jnp.where(deg > 0, lax.rsqrt(jnp.maximum(deg, 1e-30)), 0.0)
    o_ref[...] = h2 * ns

  return pl.pallas_call(
      body,
      out_shape=jax.ShapeDtypeStruct((N_S, 128), jnp.float32),
  )(hs_new, Wp, bp, W2, hs2)


def _tc_final(pA, pB, tab1, hd1a, hd1b, q2, hd2, ov, b1, b2, g1, be1, g2,
              be2, Wm1a, Wm1b, bm1, Wm2, bm2, fw_row):
  def body(pa, pb, t1, h1a, h1b, qa, h2, ovr, b1r, b2r, g1r, be1r, g2r,
           be2r, wa, wb, bm1r, w2r, bm2r, fwr, outr, x1s, x2s, sts):
    i = pl.program_id(0)

    @pl.when(i < 5)
    def _():
      x1 = (pa[...] + pb[...] + t1[...]) * lax.rsqrt(
          h1a[...] + h1b[...] + 1.0) + b1r[...]
      d2 = h2[...]
      nd2 = jnp.where(d2 > 0, lax.rsqrt(jnp.maximum(d2, 1e-30)), 0.0)
      x2 = qa[...] * nd2 + b2r[...]
      r = pl.ds((i % 5) * 2000, 2000)
      x1s[r, :] = x1
      x2s[r, :] = x2
      st = jnp.concatenate([
          jnp.sum(x1, 0, keepdims=True), jnp.sum(x1 * x1, 0, keepdims=True),
          jnp.sum(x2, 0, keepdims=True), jnp.sum(x2 * x2, 0, keepdims=True),
          jnp.zeros((4, 128), jnp.float32)], 0)

      @pl.when(i == 0)
      def _():
        sts[...] = st

      @pl.when(i != 0)
      def _():
        sts[...] += st

    @pl.when(i >= 5)
    def _():
      st = sts[...]
      n = float(N_T)
      r = pl.ds((i % 5) * 2000, 2000)
      mu1, q1 = st[0:1] / n, st[1:2] / n
      var1 = q1 - mu1 * mu1
      H1 = _leaky((x1s[r, :] - mu1) * lax.rsqrt(var1 + 1e-5) * g1r[...]
                  + be1r[...])
      mu2, qq2 = st[2:3] / n, st[3:4] / n
      var2 = qq2 - mu2 * mu2
      no = _leaky((x2s[r, :] - mu2) * lax.rsqrt(var2 + 1e-5) * g2r[...]
                  + be2r[...])
      m = _leaky(
          jnp.dot(no, wa[...], preferred_element_type=jnp.float32)
          + jnp.dot(ovr[...], wb[...], preferred_element_type=jnp.float32)
          + bm1r[...])
      Hi = jnp.dot(m, w2r[...], preferred_element_type=jnp.float32) + bm2r[...]
      fwv = fwr[...]
      lane = lax.broadcasted_iota(jnp.int32, (1, 128), 1)
      msk = lane < 2
      mx = jnp.max(jnp.where(msk, fwv, -jnp.inf))
      e = jnp.where(msk, jnp.exp(fwv - mx), 0.0)
      ssum = jnp.sum(e)
      w0 = jnp.sum(jnp.where(lane == 0, e, 0.0)) / ssum
      w1 = jnp.sum(jnp.where(lane == 1, e, 0.0)) / ssum
      outr[...] = w0 * H1 + w1 * Hi

  blk = lambda c: pl.BlockSpec((2000, c), lambda i: (i % 5, 0))
  full = lambda r, c: pl.BlockSpec((r, c), lambda i: (0, 0))
  return pl.pallas_call(
      body,
      grid=(10,),
      in_specs=[blk(128), blk(128), blk(128), blk(1), blk(1), blk(128),
                blk(1), blk(128), full(1, 128), full(1, 128), full(1, 128),
                full(1, 128), full(1, 128), full(1, 128), full(128, 256),
                full(128, 256), full(1, 256), full(256, 128), full(1, 128),
                full(1, 128)],
      out_specs=blk(128),
      out_shape=jax.ShapeDtypeStruct((N_T, 128), jnp.float32),
      scratch_shapes=[
          pltpu.VMEM((N_T, 128), jnp.float32),
          pltpu.VMEM((N_T, 128), jnp.float32),
          pltpu.VMEM((8, 128), jnp.float32),
      ],
  )(pA, pB, tab1, hd1a, hd1b, q2, hd2, ov, b1, b2, g1, be1, g2, be2, Wm1a,
    Wm1b, bm1, Wm2, bm2, fw_row)


def _pad_edges(src, dst, e_pad, src_junk_base, n_src_junk, dst_junk_base,
               n_dst_junk):
  e = src.shape[0]
  npad = e_pad - e
  r = jnp.arange(npad, dtype=jnp.int32)
  sp = jnp.concatenate([src, src_junk_base + r % n_src_junk])
  dp = jnp.concatenate([dst, dst_junk_base + r % n_dst_junk])
  return sp.reshape(-1, 128), dp.reshape(-1, 128)


def kernel(feat, hs_new_feat, hs_pool_feat, W1, b1, g1, be1, Wp, bp, W2, b2,
           g2, be2, Wm1, bm1, Wm2, bm2, fusion_weights, intra_edge_index,
           inter_edge_index, pool_edge_index):
  f32 = jnp.float32
  si1, di1 = _pad_edges(intra_edge_index[0], intra_edge_index[1], E1P,
                        N_T, N_TP - N_T, N_T, N_TP - N_T)
  si2, di2 = _pad_edges(inter_edge_index[0], inter_edge_index[1], E2P,
                        N_S, N_SP - N_S, N_T, N_TP - N_T)
  sip, dip = _pad_edges(pool_edge_index[0], pool_edge_index[1], EPP,
                        N_S, N_SP - N_S, N_T, N_TP - N_T)
  si2 = si2.reshape(16, 20, 128)
  di2 = di2.reshape(16, 20, 128)
  sip = sip.reshape(16, 6, 128)
  dip = dip.reshape(16, 6, 128)
  s1m = si1[:2304].reshape(32, 72, 128)
  d1m = di1[:2304].reshape(32, 72, 128)
  s1e = si1[2304:].reshape(16, 16, 128)
  d1e = di1[2304:].reshape(16, 16, 128)
  s1a = si1[:1408].reshape(16, 88, 128)
  d1a = di1[:1408].reshape(16, 88, 128)
  s1b = si1[1408:].reshape(16, 72, 128)
  d1b = di1[1408:].reshape(16, 72, 128)
  pool_tab = jnp.concatenate(
      [hs_pool_feat, jnp.zeros((N_SP - N_S, 128), f32)], 0)
  zeros128 = jnp.zeros((N_TP, 128), f32)
  iota128 = jnp.arange(128, dtype=jnp.int32).reshape(1, 128)

  hs1, hd1, hs2, hd2 = _sc_hists(s1a, d1a, s1b, d1b, si2, di2, zeros128,
                                 iota128)
  flat = lambda a, n: a.reshape(-1)[:n].reshape(n, 1)
  hs1a, hs1b = flat(hs1[0], N_T), flat(hs1[1], N_T)
  hd1a, hd1b = flat(hd1[0], N_T), flat(hd1[1], N_T)
  hs2f = flat(hs2, N_S)
  hd2f = flat(hd2, N_T)

  tab1 = _tc_tab1(feat, W1, hs1a, hs1b)
  tab2 = _tc_tab2(hs_new_feat, Wp, bp.reshape(1, 128), W2, hs2f)
  tab1p = jnp.concatenate([tab1, jnp.zeros((N_TP - N_T, 128), f32)], 0)
  tab2p = jnp.concatenate([tab2, jnp.zeros((N_SP - N_S, 128), f32)], 0)

  p1, q2, ov = _sc_agg(tab1p, tab2p, pool_tab, s1m, d1m, s1e, d1e, si2, di2,
                       sip, dip, zeros128)

  fw_row = jnp.zeros((1, 128), f32).at[0, :2].set(fusion_weights[0])
  out = _tc_final(p1[0, :N_T], p1[1, :N_T], tab1, hd1a, hd1b, q2[:N_T],
                  hd2f, ov[:N_T], b1.reshape(1, 128), b2.reshape(1, 128),
                  g1.reshape(1, 128), be1.reshape(1, 128),
                  g2.reshape(1, 128), be2.reshape(1, 128), Wm1[:128],
                  Wm1[128:], bm1.reshape(1, 256), Wm2, bm2.reshape(1, 128),
                  fw_row)
  return out


# speedup vs baseline: 1.1620x; 1.1620x over previous
import dataclasses
import functools

import jax
import jax.numpy as jnp
from jax import lax
from jax.experimental import pallas as pl
from jax.experimental.pallas import tpu as pltpu
from jax.experimental.pallas import tpu_sc as plsc

N_T = 10000
N_S = 2500
N_TP = 10240
N_SP = 2560
E1P = 327680
E2P = 40960
EPP = 12288
R1 = E1P // 128
R2 = E2P // 128
RP = EPP // 128

_mesh = plsc.VectorSubcoreMesh(core_axis_name="c", subcore_axis_name="s")


_cp = pltpu.CompilerParams()
if "needs_layout_passes" in pltpu.CompilerParams.__dataclass_fields__:
  _cp = dataclasses.replace(_cp, needs_layout_passes=False)


def _sc_hists(si1, di1, si2, di2, zeros128, iota128):
  @functools.partial(
      pl.kernel,
      out_type=(
          jax.ShapeDtypeStruct((128, 128), jnp.float32),
          jax.ShapeDtypeStruct((128, 128), jnp.float32),
          jax.ShapeDtypeStruct((128, 128), jnp.float32),
          jax.ShapeDtypeStruct((128, 128), jnp.float32),
      ),
      mesh=_mesh,
      scratch_types=[
          pltpu.VMEM_SHARED((128, 128), jnp.float32),
          pltpu.VMEM_SHARED((128, 128), jnp.float32),
          pltpu.VMEM((160, 128), jnp.int32),
          pltpu.VMEM((20, 128), jnp.int32),
          pltpu.VMEM((128, 128), jnp.float32),
          pltpu.VMEM((128, 128), jnp.float32),
          pltpu.VMEM((1, 128), jnp.int32),
      ],
      compiler_params=_cp,
  )
  def k(si1_h, di1_h, si2_h, di2_h, z128_h, iota_h, o_hs1, o_hd1, o_hs2,
        o_hd2, HsA, HsB, idx_s, idx_d, localA, localB, iota_v):
    ci = lax.axis_index("c")
    ti = lax.axis_index("s")
    z128 = pl.ds(0, 128)
    sl8 = pl.ds(ti * 8, 8)
    pltpu.sync_copy(z128_h.at[z128], localA)
    pltpu.sync_copy(z128_h.at[z128], localB)
    pltpu.sync_copy(z128_h.at[sl8], HsA.at[sl8])
    pltpu.sync_copy(z128_h.at[sl8], HsB.at[sl8])
    pltpu.sync_copy(iota_h, iota_v)
    ones = jnp.ones((16,), jnp.float32)

    def count(idx_ref, local_ref, nrows):
      @pl.loop(0, nrows)
      def _(j):
        for c in range(0, 128, 16):
          x = idx_ref[j, pl.ds(c, 16)]
          row = lax.shift_right_logical(x, 7)
          col = lax.bitwise_and(x, 127)
          plsc.addupdate_scatter(local_ref, [row, col], ones)

    @pl.when(ci == 0)
    def _():
      pltpu.sync_copy(si1_h.at[pl.ds(ti * 160, 160)], idx_s)
      pltpu.sync_copy(si2_h.at[ti], idx_d)

    @pl.when(ci == 1)
    def _():
      pltpu.sync_copy(di1_h.at[pl.ds(ti * 160, 160)], idx_s)
      pltpu.sync_copy(di2_h.at[ti], idx_d)

    count(idx_s, localA, 160)
    count(idx_d, localB, 20)
    plsc.subcore_barrier()
    pltpu.sync_copy(localA, HsA.at[iota_v.at[0]], add=True)
    pltpu.sync_copy(localB, HsB.at[iota_v.at[0]], add=True)
    plsc.subcore_barrier()

    @pl.when(ci == 0)
    def _():
      pltpu.sync_copy(HsA.at[sl8], o_hs1.at[sl8])
      pltpu.sync_copy(HsB.at[sl8], o_hs2.at[sl8])

    @pl.when(ci == 1)
    def _():
      pltpu.sync_copy(HsA.at[sl8], o_hd1.at[sl8])
      pltpu.sync_copy(HsB.at[sl8], o_hd2.at[sl8])

  return k(si1, di1, si2, di2, zeros128, iota128)


def _sc_agg(tab1, tab2, ptab, s1m, d1m, s1e, d1e, s2, d2, sp, dp, zeros128):
  @functools.partial(
      pl.kernel,
      out_type=(
          jax.ShapeDtypeStruct((N_TP, 128), jnp.float32),
          jax.ShapeDtypeStruct((N_TP, 128), jnp.float32),
          jax.ShapeDtypeStruct((N_TP, 128), jnp.float32),
          jax.ShapeDtypeStruct((N_TP, 128), jnp.float32),
      ),
      mesh=_mesh,
      scratch_types=[
          pltpu.VMEM_SHARED((N_TP, 128), jnp.float32),
          pltpu.VMEM((16, 128), jnp.int32),
          pltpu.VMEM((16, 128), jnp.int32),
          pltpu.VMEM((128, 128), jnp.float32),
          pltpu.VMEM((128, 128), jnp.float32),
          pltpu.SemaphoreType.DMA,
          pltpu.SemaphoreType.DMA,
          pltpu.SemaphoreType.DMA,
          pltpu.SemaphoreType.DMA,
      ],
  )
  def k(tab1_h, tab2_h, ptab_h, s1m_h, d1m_h, s1e_h, d1e_h, s2_h, d2_h, sp_h,
        dp_h, z128_h, o_p1a, o_p1b, o_q2, o_ov, acc, idx_s, idx_d, rows0, rows1,
        gs0, gs1, ss0, ss1):
    ci = lax.axis_index("c")
    ti = lax.axis_index("s")
    sl = pl.ds(ti * 640, 640)

    def chunk(tab_h, s3_h, d3_h, widx, r0, n):
      pltpu.sync_copy(s3_h.at[widx, pl.ds(r0, n)], idx_s.at[pl.ds(0, n)])
      pltpu.sync_copy(d3_h.at[widx, pl.ds(r0, n)], idx_d.at[pl.ds(0, n)])

      @pl.loop(0, n, step=2)
      def _(j):
        g0 = pltpu.async_copy(tab_h.at[idx_s.at[j]], rows0, gs0)
        g1 = pltpu.async_copy(tab_h.at[idx_s.at[j + 1]], rows1, gs1)
        g0.wait()
        s0 = pltpu.async_copy(rows0, acc.at[idx_d.at[j]], ss0, add=True)
        g1.wait()
        s1 = pltpu.async_copy(rows1, acc.at[idx_d.at[j + 1]], ss1, add=True)
        s0.wait()
        s1.wait()

    pltpu.sync_copy(z128_h.at[sl], acc.at[sl])
    plsc.subcore_barrier()
    w = ci * 16 + ti

    @pl.loop(0, 72, step=8)
    def _(r):
      chunk(tab1_h, s1m_h, d1m_h, w, r, 8)

    @pl.when(ci == 0)
    def _():
      chunk(tab1_h, s1e_h, d1e_h, ti, 0, 16)

    plsc.subcore_barrier()

    @pl.when(ci == 0)
    def _():
      pltpu.sync_copy(acc.at[sl], o_p1a.at[sl])

    @pl.when(ci == 1)
    def _():
      pltpu.sync_copy(acc.at[sl], o_p1b.at[sl])

    plsc.subcore_barrier()

    pltpu.sync_copy(z128_h.at[sl], acc.at[sl])
    plsc.subcore_barrier()

    @pl.when(ci == 1)
    def _():
      chunk(tab2_h, s2_h, d2_h, ti, 0, 16)
      chunk(tab2_h, s2_h, d2_h, ti, 16, 4)
      plsc.subcore_barrier()
      pltpu.sync_copy(acc.at[sl], o_q2.at[sl])

    @pl.when(ci == 0)
    def _():
      chunk(ptab_h, sp_h, dp_h, ti, 0, 6)
      plsc.subcore_barrier()
      pltpu.sync_copy(acc.at[sl], o_ov.at[sl])

  return k(tab1, tab2, ptab, s1m, d1m, s1e, d1e, s2, d2, sp, dp, zeros128)


def _leaky(x):
  return jnp.where(x >= 0, x, 0.01 * x)


def _tc_tab1(feat, W1, hs1):
  def body(f_ref, w_ref, h_ref, o_ref):
    h = jnp.dot(f_ref[...], w_ref[...], preferred_element_type=jnp.float32)
    o_ref[...] = h * lax.rsqrt(h_ref[...] + 1.0)

  return pl.pallas_call(
      body,
      grid=(5,),
      in_specs=[
          pl.BlockSpec((2048, 128), lambda i: (i, 0)),
          pl.BlockSpec((128, 128), lambda i: (0, 0)),
          pl.BlockSpec((2048, 1), lambda i: (i, 0)),
      ],
      out_specs=pl.BlockSpec((2048, 128), lambda i: (i, 0)),
      out_shape=jax.ShapeDtypeStruct((N_TP, 128), jnp.float32),
  )(feat, W1, hs1)


def _tc_tab2(hs_new, Wp, bp, W2, hs2):
  def body(x_ref, wp_ref, bp_ref, w2_ref, h_ref, o_ref):
    fm = jnp.dot(x_ref[...], wp_ref[...],
                 preferred_element_type=jnp.float32) + bp_ref[...]
    h2 = jnp.dot(fm, w2_ref[...], preferred_element_type=jnp.float32)
    deg = h_ref[...]
    ns = jnp.where(deg > 0, lax.rsqrt(jnp.maximum(deg, 1e-30)), 0.0)
    o_ref[...] = h2 * ns

  return pl.pallas_call(
      body,
      in_specs=[
          pl.BlockSpec((N_SP, 128), lambda: (0, 0)),
          pl.BlockSpec((128, 128), lambda: (0, 0)),
          pl.BlockSpec((1, 128), lambda: (0, 0)),
          pl.BlockSpec((128, 128), lambda: (0, 0)),
          pl.BlockSpec((N_SP, 1), lambda: (0, 0)),
      ],
      out_specs=pl.BlockSpec((N_SP, 128), lambda: (0, 0)),
      out_shape=jax.ShapeDtypeStruct((N_SP, 128), jnp.float32),
  )(hs_new, Wp, bp, W2, hs2)


def _tc_final(pA, pB, tab1, hd1, q2, hd2, ov, b1, b2, g1, be1, g2,
              be2, Wm1a, Wm1b, bm1, Wm2, bm2, fw_row):
  def body(pa, pb, t1, h1, qa, h2, ovr, b1r, b2r, g1r, be1r, g2r,
           be2r, wa, wb, bm1r, w2r, bm2r, fwr, outr, x1s, x2s, sts):
    i = pl.program_id(0)

    @pl.when(i < 5)
    def _():
      x1 = (pa[...] + pb[...] + t1[...]) * lax.rsqrt(h1[...] + 1.0) + b1r[...]
      d2 = h2[...]
      nd2 = jnp.where(d2 > 0, lax.rsqrt(jnp.maximum(d2, 1e-30)), 0.0)
      x2 = qa[...] * nd2 + b2r[...]
      r = pl.ds((i % 5) * 2000, 2000)
      x1s[r, :] = x1
      x2s[r, :] = x2
      st = jnp.concatenate([
          jnp.sum(x1, 0, keepdims=True), jnp.sum(x1 * x1, 0, keepdims=True),
          jnp.sum(x2, 0, keepdims=True), jnp.sum(x2 * x2, 0, keepdims=True),
          jnp.zeros((4, 128), jnp.float32)], 0)

      @pl.when(i == 0)
      def _():
        sts[...] = st

      @pl.when(i != 0)
      def _():
        sts[...] += st

    @pl.when(i >= 5)
    def _():
      st = sts[...]
      n = float(N_T)
      r = pl.ds((i % 5) * 2000, 2000)
      mu1, q1 = st[0:1] / n, st[1:2] / n
      var1 = q1 - mu1 * mu1
      H1 = _leaky((x1s[r, :] - mu1) * lax.rsqrt(var1 + 1e-5) * g1r[...]
                  + be1r[...])
      mu2, qq2 = st[2:3] / n, st[3:4] / n
      var2 = qq2 - mu2 * mu2
      no = _leaky((x2s[r, :] - mu2) * lax.rsqrt(var2 + 1e-5) * g2r[...]
                  + be2r[...])
      m = _leaky(
          jnp.dot(no, wa[...], preferred_element_type=jnp.float32)
          + jnp.dot(ovr[...], wb[...], preferred_element_type=jnp.float32)
          + bm1r[...])
      Hi = jnp.dot(m, w2r[...], preferred_element_type=jnp.float32) + bm2r[...]
      fwv = fwr[...]
      lane = lax.broadcasted_iota(jnp.int32, (1, 128), 1)
      msk = lane < 2
      mx = jnp.max(jnp.where(msk, fwv, -jnp.inf))
      e = jnp.where(msk, jnp.exp(fwv - mx), 0.0)
      ssum = jnp.sum(e)
      w0 = jnp.sum(jnp.where(lane == 0, e, 0.0)) / ssum
      w1 = jnp.sum(jnp.where(lane == 1, e, 0.0)) / ssum
      outr[...] = w0 * H1 + w1 * Hi

  p1b = lambda c: pl.BlockSpec((2000, c), lambda i: (jnp.minimum(i, 4), 0))
  p2b = lambda c: pl.BlockSpec((2000, c),
                               lambda i: (jnp.maximum(i, 5) - 5, 0))
  full = lambda r, c: pl.BlockSpec((r, c), lambda i: (0, 0))
  return pl.pallas_call(
      body,
      grid=(10,),
      in_specs=[p1b(128), p1b(128), p1b(128), p1b(1), p1b(128),
                p1b(1), p2b(128), full(1, 128), full(1, 128), full(1, 128),
                full(1, 128), full(1, 128), full(1, 128), full(128, 256),
                full(128, 256), full(1, 256), full(256, 128), full(1, 128),
                full(1, 128)],
      out_specs=p2b(128),
      out_shape=jax.ShapeDtypeStruct((N_T, 128), jnp.float32),
      scratch_shapes=[
          pltpu.VMEM((N_T, 128), jnp.float32),
          pltpu.VMEM((N_T, 128), jnp.float32),
          pltpu.VMEM((8, 128), jnp.float32),
      ],
  )(pA, pB, tab1, hd1, q2, hd2, ov, b1, b2, g1, be1, g2, be2, Wm1a,
    Wm1b, bm1, Wm2, bm2, fw_row)


def _pad_edges(src, dst, e_pad, src_junk_base, n_src_junk, dst_junk_base,
               n_dst_junk):
  e = src.shape[0]
  npad = e_pad - e
  r = jnp.arange(npad, dtype=jnp.int32)
  sp = jnp.concatenate([src, src_junk_base + r % n_src_junk])
  dp = jnp.concatenate([dst, dst_junk_base + r % n_dst_junk])
  return sp.reshape(-1, 128), dp.reshape(-1, 128)


def kernel(feat, hs_new_feat, hs_pool_feat, W1, b1, g1, be1, Wp, bp, W2, b2,
           g2, be2, Wm1, bm1, Wm2, bm2, fusion_weights, intra_edge_index,
           inter_edge_index, pool_edge_index):
  f32 = jnp.float32
  si1, di1 = _pad_edges(intra_edge_index[0], intra_edge_index[1], E1P,
                        N_T, N_TP - N_T, N_T, N_TP - N_T)
  si2, di2 = _pad_edges(inter_edge_index[0], inter_edge_index[1], E2P,
                        N_S, N_SP - N_S, N_T, N_TP - N_T)
  sip, dip = _pad_edges(pool_edge_index[0], pool_edge_index[1], EPP,
                        N_S, N_SP - N_S, N_T, N_TP - N_T)
  si2_3 = si2.reshape(16, 20, 128)
  di2_3 = di2.reshape(16, 20, 128)
  sip = sip.reshape(16, 6, 128)
  dip = dip.reshape(16, 6, 128)
  s1m = si1[:2304].reshape(32, 72, 128)
  d1m = di1[:2304].reshape(32, 72, 128)
  s1e = si1[2304:].reshape(16, 16, 128)
  d1e = di1[2304:].reshape(16, 16, 128)
  pool_tab = jnp.concatenate(
      [hs_pool_feat, jnp.zeros((N_SP - N_S, 128), f32)], 0)
  zeros128 = jnp.zeros((N_TP, 128), f32)
  iota128 = jnp.arange(128, dtype=jnp.int32).reshape(1, 128)

  hs1, hd1, hs2, hd2 = _sc_hists(si1, di1, si2_3, di2_3, zeros128, iota128)
  flat = lambda a, n: a.reshape(-1)[:n].reshape(n, 1)
  hs1f = flat(hs1, N_T)
  hd1f = flat(hd1, N_T)
  hs2f = flat(hs2, N_SP)
  hd2f = flat(hd2, N_T)

  tab1p = _tc_tab1(feat, W1, hs1f)
  hs_new_p = jnp.concatenate(
      [hs_new_feat, jnp.zeros((N_SP - N_S, 128), f32)], 0)
  tab2p = _tc_tab2(hs_new_p, Wp, bp.reshape(1, 128), W2, hs2f)

  p1a, p1b_, q2, ov = _sc_agg(tab1p, tab2p, pool_tab, s1m, d1m, s1e, d1e,
                              si2_3, di2_3, sip, dip, zeros128)

  fw_row = jnp.zeros((1, 128), f32).at[0, :2].set(fusion_weights[0])
  out = _tc_final(p1a, p1b_, tab1p, hd1f, q2, hd2f, ov,
                  b1.reshape(1, 128), b2.reshape(1, 128),
                  g1.reshape(1, 128), be1.reshape(1, 128),
                  g2.reshape(1, 128), be2.reshape(1, 128), Wm1[:128],
                  Wm1[128:], bm1.reshape(1, 256), Wm2, bm2.reshape(1, 128),
                  fw_row)
  return out


# speedup vs baseline: 1.1629x; 1.0008x over previous
import dataclasses
import functools

import jax
import jax.numpy as jnp
from jax import lax
from jax.experimental import pallas as pl
from jax.experimental.pallas import tpu as pltpu
from jax.experimental.pallas import tpu_sc as plsc

N_T = 10000
N_S = 2500
N_TP = 10240
N_SP = 2560
E1P = 327680
E2P = 40960
EPP = 12288
R1 = E1P // 128
R2 = E2P // 128
RP = EPP // 128

_mesh = plsc.VectorSubcoreMesh(core_axis_name="c", subcore_axis_name="s")


_cp = pltpu.CompilerParams()
if "needs_layout_passes" in pltpu.CompilerParams.__dataclass_fields__:
  _cp = dataclasses.replace(_cp, needs_layout_passes=False)


def _sc_hists(si1, di1, si2, di2, zeros128, iota128):
  @functools.partial(
      pl.kernel,
      out_type=(
          jax.ShapeDtypeStruct((128, 128), jnp.float32),
          jax.ShapeDtypeStruct((128, 128), jnp.float32),
          jax.ShapeDtypeStruct((128, 128), jnp.float32),
          jax.ShapeDtypeStruct((128, 128), jnp.float32),
      ),
      mesh=_mesh,
      scratch_types=[
          pltpu.VMEM_SHARED((128, 128), jnp.float32),
          pltpu.VMEM_SHARED((128, 128), jnp.float32),
          pltpu.VMEM((160, 128), jnp.int32),
          pltpu.VMEM((20, 128), jnp.int32),
          pltpu.VMEM((128, 128), jnp.float32),
          pltpu.VMEM((128, 128), jnp.float32),
          pltpu.VMEM((1, 128), jnp.int32),
      ],
      compiler_params=_cp,
  )
  def k(si1_h, di1_h, si2_h, di2_h, z128_h, iota_h, o_hs1, o_hd1, o_hs2,
        o_hd2, HsA, HsB, idx_s, idx_d, localA, localB, iota_v):
    ci = lax.axis_index("c")
    ti = lax.axis_index("s")
    z128 = pl.ds(0, 128)
    sl8 = pl.ds(ti * 8, 8)
    pltpu.sync_copy(z128_h.at[z128], localA)
    pltpu.sync_copy(z128_h.at[z128], localB)
    pltpu.sync_copy(z128_h.at[sl8], HsA.at[sl8])
    pltpu.sync_copy(z128_h.at[sl8], HsB.at[sl8])
    pltpu.sync_copy(iota_h, iota_v)
    ones = jnp.ones((16,), jnp.float32)

    def count(idx_ref, local_ref, nrows):
      @pl.loop(0, nrows)
      def _(j):
        for c in range(0, 128, 16):
          x = idx_ref[j, pl.ds(c, 16)]
          row = lax.shift_right_logical(x, 7)
          col = lax.bitwise_and(x, 127)
          plsc.addupdate_scatter(local_ref, [row, col], ones)

    @pl.when(ci == 0)
    def _():
      pltpu.sync_copy(si1_h.at[pl.ds(ti * 160, 160)], idx_s)
      pltpu.sync_copy(si2_h.at[ti], idx_d)

    @pl.when(ci == 1)
    def _():
      pltpu.sync_copy(di1_h.at[pl.ds(ti * 160, 160)], idx_s)
      pltpu.sync_copy(di2_h.at[ti], idx_d)

    count(idx_s, localA, 160)
    count(idx_d, localB, 20)
    plsc.subcore_barrier()
    pltpu.sync_copy(localA, HsA.at[iota_v.at[0]], add=True)
    pltpu.sync_copy(localB, HsB.at[iota_v.at[0]], add=True)
    plsc.subcore_barrier()

    @pl.when(ci == 0)
    def _():
      pltpu.sync_copy(HsA.at[sl8], o_hs1.at[sl8])
      pltpu.sync_copy(HsB.at[sl8], o_hs2.at[sl8])

    @pl.when(ci == 1)
    def _():
      pltpu.sync_copy(HsA.at[sl8], o_hd1.at[sl8])
      pltpu.sync_copy(HsB.at[sl8], o_hd2.at[sl8])

  return k(si1, di1, si2, di2, zeros128, iota128)


def _sc_agg(tab1, tab2, ptab, s1m, d1m, s1e, d1e, s2, d2, sp, dp, zeros128):
  @functools.partial(
      pl.kernel,
      out_type=(
          jax.ShapeDtypeStruct((N_TP, 128), jnp.float32),
          jax.ShapeDtypeStruct((N_TP, 128), jnp.float32),
          jax.ShapeDtypeStruct((N_TP, 128), jnp.float32),
          jax.ShapeDtypeStruct((N_TP, 128), jnp.float32),
      ),
      mesh=_mesh,
      scratch_types=[
          pltpu.VMEM_SHARED((N_TP, 128), jnp.float32),
          pltpu.VMEM((16, 128), jnp.int32),
          pltpu.VMEM((16, 2, 64), jnp.int32),
          pltpu.VMEM((128, 128), jnp.float32),
          pltpu.VMEM((128, 128), jnp.float32),
          pltpu.SemaphoreType.DMA,
          pltpu.SemaphoreType.DMA,
          pltpu.SemaphoreType.DMA,
          pltpu.SemaphoreType.DMA,
          pltpu.SemaphoreType.DMA,
          pltpu.SemaphoreType.DMA,
          pltpu.SemaphoreType.DMA,
          pltpu.SemaphoreType.DMA,
      ],
  )
  def k(tab1_h, tab2_h, ptab_h, s1m_h, d1m_h, s1e_h, d1e_h, s2_h, d2_h, sp_h,
        dp_h, z128_h, o_p1a, o_p1b, o_q2, o_ov, acc, idx_s, idx_d, rows0,
        rows1, gs0a, gs0b, gs1a, gs1b, ss0a, ss0b, ss1a, ss1b):
    ci = lax.axis_index("c")
    ti = lax.axis_index("s")
    sl = pl.ds(ti * 640, 640)
    lo = pl.ds(0, 64)
    hi = pl.ds(64, 64)

    def chunk(tab_h, s3_h, d3_h, widx, r0, n):
      pltpu.sync_copy(s3_h.at[widx, pl.ds(r0, n)], idx_s.at[pl.ds(0, n)])
      pltpu.sync_copy(d3_h.at[widx, pl.ds(r0, n)], idx_d.at[pl.ds(0, n)])

      @pl.loop(0, n, step=2)
      def _(j):
        g0a = pltpu.async_copy(tab_h.at[idx_s.at[j, lo]], rows0.at[lo], gs0a)
        g0b = pltpu.async_copy(tab_h.at[idx_s.at[j, hi]], rows0.at[hi], gs0b)
        g1a = pltpu.async_copy(tab_h.at[idx_s.at[j + 1, lo]], rows1.at[lo],
                               gs1a)
        g1b = pltpu.async_copy(tab_h.at[idx_s.at[j + 1, hi]], rows1.at[hi],
                               gs1b)
        g0a.wait()
        s0a = pltpu.async_copy(rows0.at[lo], acc.at[idx_d.at[j, 0]], ss0a,
                               add=True)
        g0b.wait()
        s0b = pltpu.async_copy(rows0.at[hi], acc.at[idx_d.at[j, 1]], ss0b,
                               add=True)
        g1a.wait()
        s1a = pltpu.async_copy(rows1.at[lo], acc.at[idx_d.at[j + 1, 0]], ss1a,
                               add=True)
        g1b.wait()
        s1b = pltpu.async_copy(rows1.at[hi], acc.at[idx_d.at[j + 1, 1]], ss1b,
                               add=True)
        s0a.wait()
        s0b.wait()
        s1a.wait()
        s1b.wait()

    pltpu.sync_copy(z128_h.at[sl], acc.at[sl])
    plsc.subcore_barrier()
    w = ci * 16 + ti

    @pl.loop(0, 72, step=8)
    def _(r):
      chunk(tab1_h, s1m_h, d1m_h, w, r, 8)

    @pl.when(ci == 0)
    def _():
      chunk(tab1_h, s1e_h, d1e_h, ti, 0, 16)

    plsc.subcore_barrier()

    @pl.when(ci == 0)
    def _():
      pltpu.sync_copy(acc.at[sl], o_p1a.at[sl])

    @pl.when(ci == 1)
    def _():
      pltpu.sync_copy(acc.at[sl], o_p1b.at[sl])

    plsc.subcore_barrier()

    pltpu.sync_copy(z128_h.at[sl], acc.at[sl])
    plsc.subcore_barrier()

    @pl.when(ci == 1)
    def _():
      chunk(tab2_h, s2_h, d2_h, ti, 0, 16)
      chunk(tab2_h, s2_h, d2_h, ti, 16, 4)
      plsc.subcore_barrier()
      pltpu.sync_copy(acc.at[sl], o_q2.at[sl])

    @pl.when(ci == 0)
    def _():
      chunk(ptab_h, sp_h, dp_h, ti, 0, 6)
      plsc.subcore_barrier()
      pltpu.sync_copy(acc.at[sl], o_ov.at[sl])

  return k(tab1, tab2, ptab, s1m, d1m, s1e, d1e, s2, d2, sp, dp, zeros128)


def _leaky(x):
  return jnp.where(x >= 0, x, 0.01 * x)


def _tc_tab1(feat, W1, hs1):
  def body(f_ref, w_ref, h_ref, o_ref):
    h = jnp.dot(f_ref[...], w_ref[...], preferred_element_type=jnp.float32)
    o_ref[...] = h * lax.rsqrt(h_ref[...] + 1.0)

  return pl.pallas_call(
      body,
      grid=(5,),
      in_specs=[
          pl.BlockSpec((2048, 128), lambda i: (i, 0)),
          pl.BlockSpec((128, 128), lambda i: (0, 0)),
          pl.BlockSpec((2048, 1), lambda i: (i, 0)),
      ],
      out_specs=pl.BlockSpec((2048, 128), lambda i: (i, 0)),
      out_shape=jax.ShapeDtypeStruct((N_TP, 128), jnp.float32),
  )(feat, W1, hs1)


def _tc_tab2(hs_new, Wp, bp, W2, hs2):
  def body(x_ref, wp_ref, bp_ref, w2_ref, h_ref, o_ref):
    fm = jnp.dot(x_ref[...], wp_ref[...],
                 preferred_element_type=jnp.float32) + bp_ref[...]
    h2 = jnp.dot(fm, w2_ref[...], preferred_element_type=jnp.float32)
    deg = h_ref[...]
    ns = jnp.where(deg > 0, lax.rsqrt(jnp.maximum(deg, 1e-30)), 0.0)
    o_ref[...] = h2 * ns

  return pl.pallas_call(
      body,
      in_specs=[
          pl.BlockSpec((N_SP, 128), lambda: (0, 0)),
          pl.BlockSpec((128, 128), lambda: (0, 0)),
          pl.BlockSpec((1, 128), lambda: (0, 0)),
          pl.BlockSpec((128, 128), lambda: (0, 0)),
          pl.BlockSpec((N_SP, 1), lambda: (0, 0)),
      ],
      out_specs=pl.BlockSpec((N_SP, 128), lambda: (0, 0)),
      out_shape=jax.ShapeDtypeStruct((N_SP, 128), jnp.float32),
  )(hs_new, Wp, bp, W2, hs2)


def _tc_final(pA, pB, tab1, hd1, q2, hd2, ov, b1, b2, g1, be1, g2,
              be2, Wm1a, Wm1b, bm1, Wm2, bm2, fw_row):
  def body(pa, pb, t1, h1, qa, h2, ovr, b1r, b2r, g1r, be1r, g2r,
           be2r, wa, wb, bm1r, w2r, bm2r, fwr, outr, x1s, x2s, sts):
    i = pl.program_id(0)

    @pl.when(i < 5)
    def _():
      x1 = (pa[...] + pb[...] + t1[...]) * lax.rsqrt(h1[...] + 1.0) + b1r[...]
      d2 = h2[...]
      nd2 = jnp.where(d2 > 0, lax.rsqrt(jnp.maximum(d2, 1e-30)), 0.0)
      x2 = qa[...] * nd2 + b2r[...]
      r = pl.ds((i % 5) * 2000, 2000)
      x1s[r, :] = x1
      x2s[r, :] = x2
      st = jnp.concatenate([
          jnp.sum(x1, 0, keepdims=True), jnp.sum(x1 * x1, 0, keepdims=True),
          jnp.sum(x2, 0, keepdims=True), jnp.sum(x2 * x2, 0, keepdims=True),
          jnp.zeros((4, 128), jnp.float32)], 0)

      @pl.when(i == 0)
      def _():
        sts[...] = st

      @pl.when(i != 0)
      def _():
        sts[...] += st

    @pl.when(i >= 5)
    def _():
      st = sts[...]
      n = float(N_T)
      r = pl.ds((i % 5) * 2000, 2000)
      mu1, q1 = st[0:1] / n, st[1:2] / n
      var1 = q1 - mu1 * mu1
      H1 = _leaky((x1s[r, :] - mu1) * lax.rsqrt(var1 + 1e-5) * g1r[...]
                  + be1r[...])
      mu2, qq2 = st[2:3] / n, st[3:4] / n
      var2 = qq2 - mu2 * mu2
      no = _leaky((x2s[r, :] - mu2) * lax.rsqrt(var2 + 1e-5) * g2r[...]
                  + be2r[...])
      m = _leaky(
          jnp.dot(no, wa[...], preferred_element_type=jnp.float32)
          + jnp.dot(ovr[...], wb[...], preferred_element_type=jnp.float32)
          + bm1r[...])
      Hi = jnp.dot(m, w2r[...], preferred_element_type=jnp.float32) + bm2r[...]
      fwv = fwr[...]
      lane = lax.broadcasted_iota(jnp.int32, (1, 128), 1)
      msk = lane < 2
      mx = jnp.max(jnp.where(msk, fwv, -jnp.inf))
      e = jnp.where(msk, jnp.exp(fwv - mx), 0.0)
      ssum = jnp.sum(e)
      w0 = jnp.sum(jnp.where(lane == 0, e, 0.0)) / ssum
      w1 = jnp.sum(jnp.where(lane == 1, e, 0.0)) / ssum
      outr[...] = w0 * H1 + w1 * Hi

  p1b = lambda c: pl.BlockSpec((2000, c), lambda i: (jnp.minimum(i, 4), 0))
  p2b = lambda c: pl.BlockSpec((2000, c),
                               lambda i: (jnp.maximum(i, 5) - 5, 0))
  full = lambda r, c: pl.BlockSpec((r, c), lambda i: (0, 0))
  return pl.pallas_call(
      body,
      grid=(10,),
      in_specs=[p1b(128), p1b(128), p1b(128), p1b(1), p1b(128),
                p1b(1), p2b(128), full(1, 128), full(1, 128), full(1, 128),
                full(1, 128), full(1, 128), full(1, 128), full(128, 256),
                full(128, 256), full(1, 256), full(256, 128), full(1, 128),
                full(1, 128)],
      out_specs=p2b(128),
      out_shape=jax.ShapeDtypeStruct((N_T, 128), jnp.float32),
      scratch_shapes=[
          pltpu.VMEM((N_T, 128), jnp.float32),
          pltpu.VMEM((N_T, 128), jnp.float32),
          pltpu.VMEM((8, 128), jnp.float32),
      ],
  )(pA, pB, tab1, hd1, q2, hd2, ov, b1, b2, g1, be1, g2, be2, Wm1a,
    Wm1b, bm1, Wm2, bm2, fw_row)


def _pad_edges(src, dst, e_pad, src_junk_base, n_src_junk, dst_junk_base,
               n_dst_junk):
  e = src.shape[0]
  npad = e_pad - e
  r = jnp.arange(npad, dtype=jnp.int32)
  sp = jnp.concatenate([src, src_junk_base + r % n_src_junk])
  dp = jnp.concatenate([dst, dst_junk_base + r % n_dst_junk])
  return sp.reshape(-1, 128), dp.reshape(-1, 128)


def kernel(feat, hs_new_feat, hs_pool_feat, W1, b1, g1, be1, Wp, bp, W2, b2,
           g2, be2, Wm1, bm1, Wm2, bm2, fusion_weights, intra_edge_index,
           inter_edge_index, pool_edge_index):
  f32 = jnp.float32
  si1, di1 = _pad_edges(intra_edge_index[0], intra_edge_index[1], E1P,
                        N_T, N_TP - N_T, N_T, N_TP - N_T)
  si2, di2 = _pad_edges(inter_edge_index[0], inter_edge_index[1], E2P,
                        N_S, N_SP - N_S, N_T, N_TP - N_T)
  sip, dip = _pad_edges(pool_edge_index[0], pool_edge_index[1], EPP,
                        N_S, N_SP - N_S, N_T, N_TP - N_T)
  si2_3 = si2.reshape(16, 20, 128)
  di2_3 = di2.reshape(16, 20, 128)
  di2_4 = di2.reshape(16, 20, 2, 64)
  sip = sip.reshape(16, 6, 128)
  dip = dip.reshape(16, 6, 2, 64)
  s1m = si1[:2304].reshape(32, 72, 128)
  d1m = di1[:2304].reshape(32, 72, 2, 64)
  s1e = si1[2304:].reshape(16, 16, 128)
  d1e = di1[2304:].reshape(16, 16, 2, 64)
  pool_tab = jnp.concatenate(
      [hs_pool_feat, jnp.zeros((N_SP - N_S, 128), f32)], 0)
  zeros128 = jnp.zeros((N_TP, 128), f32)
  iota128 = jnp.arange(128, dtype=jnp.int32).reshape(1, 128)

  hs1, hd1, hs2, hd2 = _sc_hists(si1, di1, si2_3, di2_3, zeros128, iota128)
  flat = lambda a, n: a.reshape(-1)[:n].reshape(n, 1)
  hs1f = flat(hs1, N_T)
  hd1f = flat(hd1, N_T)
  hs2f = flat(hs2, N_SP)
  hd2f = flat(hd2, N_T)

  tab1p = _tc_tab1(feat, W1, hs1f)
  hs_new_p = jnp.concatenate(
      [hs_new_feat, jnp.zeros((N_SP - N_S, 128), f32)], 0)
  tab2p = _tc_tab2(hs_new_p, Wp, bp.reshape(1, 128), W2, hs2f)

  p1a, p1b_, q2, ov = _sc_agg(tab1p, tab2p, pool_tab, s1m, d1m, s1e, d1e,
                              si2_3, di2_4, sip, dip, zeros128)

  fw_row = jnp.zeros((1, 128), f32).at[0, :2].set(fusion_weights[0])
  out = _tc_final(p1a, p1b_, tab1p, hd1f, q2, hd2f, ov,
                  b1.reshape(1, 128), b2.reshape(1, 128),
                  g1.reshape(1, 128), be1.reshape(1, 128),
                  g2.reshape(1, 128), be2.reshape(1, 128), Wm1[:128],
                  Wm1[128:], bm1.reshape(1, 256), Wm2, bm2.reshape(1, 128),
                  fw_row)
  return out


# speedup vs baseline: 1.1645x; 1.0014x over previous
import dataclasses
import functools

import jax
import jax.numpy as jnp
from jax import lax
from jax.experimental import pallas as pl
from jax.experimental.pallas import tpu as pltpu
from jax.experimental.pallas import tpu_sc as plsc

N_T = 10000
N_S = 2500
N_TP = 10240
N_SP = 2560
E1P = 327680
E2P = 40960
EPP = 12288
R1 = E1P // 128
R2 = E2P // 128
RP = EPP // 128

_mesh = plsc.VectorSubcoreMesh(core_axis_name="c", subcore_axis_name="s")


_cp = pltpu.CompilerParams()
if "needs_layout_passes" in pltpu.CompilerParams.__dataclass_fields__:
  _cp = dataclasses.replace(_cp, needs_layout_passes=False)


def _sc_hists(si1, di1, si2, di2, zeros128, iota128):
  @functools.partial(
      pl.kernel,
      out_type=(
          jax.ShapeDtypeStruct((128, 128), jnp.float32),
          jax.ShapeDtypeStruct((128, 128), jnp.float32),
          jax.ShapeDtypeStruct((128, 128), jnp.float32),
          jax.ShapeDtypeStruct((128, 128), jnp.float32),
      ),
      mesh=_mesh,
      scratch_types=[
          pltpu.VMEM_SHARED((128, 128), jnp.float32),
          pltpu.VMEM_SHARED((128, 128), jnp.float32),
          pltpu.VMEM((160, 128), jnp.int32),
          pltpu.VMEM((20, 128), jnp.int32),
          pltpu.VMEM((128, 128), jnp.float32),
          pltpu.VMEM((128, 128), jnp.float32),
          pltpu.VMEM((1, 128), jnp.int32),
      ],
      compiler_params=_cp,
  )
  def k(si1_h, di1_h, si2_h, di2_h, z128_h, iota_h, o_hs1, o_hd1, o_hs2,
        o_hd2, HsA, HsB, idx_s, idx_d, localA, localB, iota_v):
    ci = lax.axis_index("c")
    ti = lax.axis_index("s")
    z128 = pl.ds(0, 128)
    sl8 = pl.ds(ti * 8, 8)
    pltpu.sync_copy(z128_h.at[z128], localA)
    pltpu.sync_copy(z128_h.at[z128], localB)
    pltpu.sync_copy(z128_h.at[sl8], HsA.at[sl8])
    pltpu.sync_copy(z128_h.at[sl8], HsB.at[sl8])
    pltpu.sync_copy(iota_h, iota_v)
    ones = jnp.ones((16,), jnp.float32)

    def count(idx_ref, local_ref, nrows):
      @pl.loop(0, nrows)
      def _(j):
        for c in range(0, 128, 16):
          x = idx_ref[j, pl.ds(c, 16)]
          row = lax.shift_right_logical(x, 7)
          col = lax.bitwise_and(x, 127)
          plsc.addupdate_scatter(local_ref, [row, col], ones)

    @pl.when(ci == 0)
    def _():
      pltpu.sync_copy(si1_h.at[pl.ds(ti * 160, 160)], idx_s)
      pltpu.sync_copy(si2_h.at[ti], idx_d)

    @pl.when(ci == 1)
    def _():
      pltpu.sync_copy(di1_h.at[pl.ds(ti * 160, 160)], idx_s)
      pltpu.sync_copy(di2_h.at[ti], idx_d)

    count(idx_s, localA, 160)
    count(idx_d, localB, 20)
    plsc.subcore_barrier()
    pltpu.sync_copy(localA, HsA.at[iota_v.at[0]], add=True)
    pltpu.sync_copy(localB, HsB.at[iota_v.at[0]], add=True)
    plsc.subcore_barrier()

    @pl.when(ci == 0)
    def _():
      pltpu.sync_copy(HsA.at[sl8], o_hs1.at[sl8])
      pltpu.sync_copy(HsB.at[sl8], o_hs2.at[sl8])

    @pl.when(ci == 1)
    def _():
      pltpu.sync_copy(HsA.at[sl8], o_hd1.at[sl8])
      pltpu.sync_copy(HsB.at[sl8], o_hd2.at[sl8])

  return k(si1, di1, si2, di2, zeros128, iota128)


def _sc_agg(tab1, tab2, ptab, s1m, d1m, s1e, d1e, s2, d2, sp, dp, zeros128):
  @functools.partial(
      pl.kernel,
      out_type=(
          jax.ShapeDtypeStruct((N_TP, 128), jnp.float32),
          jax.ShapeDtypeStruct((N_TP, 128), jnp.float32),
          jax.ShapeDtypeStruct((N_TP, 128), jnp.float32),
          jax.ShapeDtypeStruct((N_TP, 128), jnp.float32),
      ),
      mesh=_mesh,
      scratch_types=[
          pltpu.VMEM_SHARED((N_TP, 128), jnp.float32),
          pltpu.VMEM((16, 128), jnp.int32),
          pltpu.VMEM((16, 128), jnp.int32),
          pltpu.VMEM((128, 128), jnp.float32),
          pltpu.VMEM((128, 128), jnp.float32),
          pltpu.SemaphoreType.DMA,
          pltpu.SemaphoreType.DMA,
          pltpu.SemaphoreType.DMA,
          pltpu.SemaphoreType.DMA,
      ],
  )
  def k(tab1_h, tab2_h, ptab_h, s1m_h, d1m_h, s1e_h, d1e_h, s2_h, d2_h, sp_h,
        dp_h, z128_h, o_p1a, o_p1b, o_q2, o_ov, acc, idx_s, idx_d, rows0, rows1,
        gs0, gs1, ss0, ss1):
    ci = lax.axis_index("c")
    ti = lax.axis_index("s")
    sl = pl.ds(ti * 640, 640)

    def chunk(tab_h, s3_h, d3_h, widx, r0, n):
      pltpu.sync_copy(s3_h.at[widx, pl.ds(r0, n)], idx_s.at[pl.ds(0, n)])
      pltpu.sync_copy(d3_h.at[widx, pl.ds(r0, n)], idx_d.at[pl.ds(0, n)])

      @pl.loop(0, n, step=2)
      def _(j):
        g0 = pltpu.async_copy(tab_h.at[idx_s.at[j]], rows0, gs0)
        g1 = pltpu.async_copy(tab_h.at[idx_s.at[j + 1]], rows1, gs1)
        g0.wait()
        s0 = pltpu.async_copy(rows0, acc.at[idx_d.at[j]], ss0, add=True)
        g1.wait()
        s1 = pltpu.async_copy(rows1, acc.at[idx_d.at[j + 1]], ss1, add=True)
        s0.wait()
        s1.wait()

    pltpu.sync_copy(z128_h.at[sl], acc.at[sl])
    plsc.subcore_barrier()
    w = ci * 16 + ti

    @pl.loop(0, 72, step=8)
    def _(r):
      chunk(tab1_h, s1m_h, d1m_h, w, r, 8)

    @pl.when(ci == 0)
    def _():
      chunk(tab1_h, s1e_h, d1e_h, ti, 0, 16)

    plsc.subcore_barrier()

    @pl.when(ci == 0)
    def _():
      pltpu.sync_copy(acc.at[sl], o_p1a.at[sl])

    @pl.when(ci == 1)
    def _():
      pltpu.sync_copy(acc.at[sl], o_p1b.at[sl])

    plsc.subcore_barrier()

    pltpu.sync_copy(z128_h.at[sl], acc.at[sl])
    plsc.subcore_barrier()

    @pl.when(ci == 1)
    def _():
      chunk(tab2_h, s2_h, d2_h, ti, 0, 16)
      chunk(tab2_h, s2_h, d2_h, ti, 16, 4)
      plsc.subcore_barrier()
      pltpu.sync_copy(acc.at[sl], o_q2.at[sl])

    @pl.when(ci == 0)
    def _():
      chunk(ptab_h, sp_h, dp_h, ti, 0, 6)
      plsc.subcore_barrier()
      pltpu.sync_copy(acc.at[sl], o_ov.at[sl])

  return k(tab1, tab2, ptab, s1m, d1m, s1e, d1e, s2, d2, sp, dp, zeros128)


def _leaky(x):
  return jnp.where(x >= 0, x, 0.01 * x)


def _tc_tab1(feat, W1, hs1):
  def body(f_ref, w_ref, h_ref, o_ref):
    h = jnp.dot(f_ref[...], w_ref[...], preferred_element_type=jnp.float32)
    o_ref[...] = h * lax.rsqrt(h_ref[...] + 1.0)

  return pl.pallas_call(
      body,
      grid=(5,),
      in_specs=[
          pl.BlockSpec((2048, 128), lambda i: (i, 0)),
          pl.BlockSpec((128, 128), lambda i: (0, 0)),
          pl.BlockSpec((2048, 1), lambda i: (i, 0)),
      ],
      out_specs=pl.BlockSpec((2048, 128), lambda i: (i, 0)),
      out_shape=jax.ShapeDtypeStruct((N_TP, 128), jnp.float32),
  )(feat, W1, hs1)


def _tc_tab2(hs_new, Wp, bp, W2, hs2):
  def body(x_ref, wp_ref, bp_ref, w2_ref, h_ref, o_ref):
    fm = jnp.dot(x_ref[...], wp_ref[...],
                 preferred_element_type=jnp.float32) + bp_ref[...]
    h2 = jnp.dot(fm, w2_ref[...], preferred_element_type=jnp.float32)
    deg = h_ref[...]
    ns = jnp.where(deg > 0, lax.rsqrt(jnp.maximum(deg, 1e-30)), 0.0)
    o_ref[...] = h2 * ns

  return pl.pallas_call(
      body,
      in_specs=[
          pl.BlockSpec((N_SP, 128), lambda: (0, 0)),
          pl.BlockSpec((128, 128), lambda: (0, 0)),
          pl.BlockSpec((1, 128), lambda: (0, 0)),
          pl.BlockSpec((128, 128), lambda: (0, 0)),
          pl.BlockSpec((N_SP, 1), lambda: (0, 0)),
      ],
      out_specs=pl.BlockSpec((N_SP, 128), lambda: (0, 0)),
      out_shape=jax.ShapeDtypeStruct((N_SP, 128), jnp.float32),
  )(hs_new, Wp, bp, W2, hs2)


def _tc_final(pA, pB, tab1, hd1, q2, hd2, ov, b1, b2, g1, be1, g2,
              be2, Wm1a, Wm1b, bm1, Wm2, bm2, fw_row):
  def body(pa, pb, t1, h1, qa, h2, ovr, b1r, b2r, g1r, be1r, g2r,
           be2r, wa, wb, bm1r, w2r, bm2r, fwr, outr, x1s, x2s, sts):
    i = pl.program_id(0)

    @pl.when(i < 5)
    def _():
      x1 = (pa[...] + pb[...] + t1[...]) * lax.rsqrt(h1[...] + 1.0) + b1r[...]
      d2 = h2[...]
      nd2 = jnp.where(d2 > 0, lax.rsqrt(jnp.maximum(d2, 1e-30)), 0.0)
      x2 = qa[...] * nd2 + b2r[...]
      r = pl.ds((i % 5) * 2000, 2000)
      x1s[r, :] = x1
      x2s[r, :] = x2
      st = jnp.concatenate([
          jnp.sum(x1, 0, keepdims=True), jnp.sum(x1 * x1, 0, keepdims=True),
          jnp.sum(x2, 0, keepdims=True), jnp.sum(x2 * x2, 0, keepdims=True),
          jnp.zeros((4, 128), jnp.float32)], 0)

      @pl.when(i == 0)
      def _():
        sts[...] = st

      @pl.when(i != 0)
      def _():
        sts[...] += st

    @pl.when(i >= 5)
    def _():
      st = sts[...]
      n = float(N_T)
      r = pl.ds((i % 5) * 2000, 2000)
      mu1, q1 = st[0:1] / n, st[1:2] / n
      var1 = q1 - mu1 * mu1
      H1 = _leaky((x1s[r, :] - mu1) * lax.rsqrt(var1 + 1e-5) * g1r[...]
                  + be1r[...])
      mu2, qq2 = st[2:3] / n, st[3:4] / n
      var2 = qq2 - mu2 * mu2
      no = _leaky((x2s[r, :] - mu2) * lax.rsqrt(var2 + 1e-5) * g2r[...]
                  + be2r[...])
      m = _leaky(
          jnp.dot(no, wa[...], preferred_element_type=jnp.float32)
          + jnp.dot(ovr[...], wb[...], preferred_element_type=jnp.float32)
          + bm1r[...])
      Hi = jnp.dot(m, w2r[...], preferred_element_type=jnp.float32) + bm2r[...]
      fwv = fwr[...]
      lane = lax.broadcasted_iota(jnp.int32, (1, 128), 1)
      msk = lane < 2
      mx = jnp.max(jnp.where(msk, fwv, -jnp.inf))
      e = jnp.where(msk, jnp.exp(fwv - mx), 0.0)
      ssum = jnp.sum(e)
      w0 = jnp.sum(jnp.where(lane == 0, e, 0.0)) / ssum
      w1 = jnp.sum(jnp.where(lane == 1, e, 0.0)) / ssum
      outr[...] = w0 * H1 + w1 * Hi

  p1b = lambda c: pl.BlockSpec((2000, c), lambda i: (jnp.minimum(i, 4), 0))
  p2b = lambda c: pl.BlockSpec((2000, c),
                               lambda i: (jnp.maximum(i, 5) - 5, 0))
  full = lambda r, c: pl.BlockSpec((r, c), lambda i: (0, 0))
  return pl.pallas_call(
      body,
      grid=(10,),
      in_specs=[p1b(128), p1b(128), p1b(128), p1b(1), p1b(128),
                p1b(1), p2b(128), full(1, 128), full(1, 128), full(1, 128),
                full(1, 128), full(1, 128), full(1, 128), full(128, 256),
                full(128, 256), full(1, 256), full(256, 128), full(1, 128),
                full(1, 128)],
      out_specs=p2b(128),
      out_shape=jax.ShapeDtypeStruct((N_T, 128), jnp.float32),
      scratch_shapes=[
          pltpu.VMEM((N_T, 128), jnp.float32),
          pltpu.VMEM((N_T, 128), jnp.float32),
          pltpu.VMEM((8, 128), jnp.float32),
      ],
  )(pA, pB, tab1, hd1, q2, hd2, ov, b1, b2, g1, be1, g2, be2, Wm1a,
    Wm1b, bm1, Wm2, bm2, fw_row)


def _pad_edges(src, dst, e_pad, src_junk_base, n_src_junk, dst_junk_base,
               n_dst_junk):
  e = src.shape[0]
  npad = e_pad - e
  r = jnp.arange(npad, dtype=jnp.int32)
  sp = jnp.concatenate([src, src_junk_base + r % n_src_junk])
  dp = jnp.concatenate([dst, dst_junk_base + r % n_dst_junk])
  return sp.reshape(-1, 128), dp.reshape(-1, 128)


def kernel(feat, hs_new_feat, hs_pool_feat, W1, b1, g1, be1, Wp, bp, W2, b2,
           g2, be2, Wm1, bm1, Wm2, bm2, fusion_weights, intra_edge_index,
           inter_edge_index, pool_edge_index):
  f32 = jnp.float32
  si1, di1 = _pad_edges(intra_edge_index[0], intra_edge_index[1], E1P,
                        N_T, N_TP - N_T, N_T, N_TP - N_T)
  si2, di2 = _pad_edges(inter_edge_index[0], inter_edge_index[1], E2P,
                        N_S, N_SP - N_S, N_T, N_TP - N_T)
  sip, dip = _pad_edges(pool_edge_index[0], pool_edge_index[1], EPP,
                        N_S, N_SP - N_S, N_T, N_TP - N_T)
  si2_3 = si2.reshape(16, 20, 128)
  di2_3 = di2.reshape(16, 20, 128)
  sip = sip.reshape(16, 6, 128)
  dip = dip.reshape(16, 6, 128)
  s1m = si1[:2304].reshape(32, 72, 128)
  d1m = di1[:2304].reshape(32, 72, 128)
  s1e = si1[2304:].reshape(16, 16, 128)
  d1e = di1[2304:].reshape(16, 16, 128)
  pool_tab = jnp.concatenate(
      [hs_pool_feat, jnp.zeros((N_SP - N_S, 128), f32)], 0)
  zeros128 = jnp.zeros((N_TP, 128), f32)
  iota128 = jnp.arange(128, dtype=jnp.int32).reshape(1, 128)

  hs1, hd1, hs2, hd2 = _sc_hists(si1, di1, si2_3, di2_3, zeros128, iota128)
  flat = lambda a, n: a.reshape(-1)[:n].reshape(n, 1)
  hs1f = flat(hs1, N_T)
  hd1f = flat(hd1, N_T)
  hs2f = flat(hs2, N_SP)
  hd2f = flat(hd2, N_T)

  tab1p = _tc_tab1(feat, W1, hs1f)
  hs_new_p = jnp.concatenate(
      [hs_new_feat, jnp.zeros((N_SP - N_S, 128), f32)], 0)
  tab2p = _tc_tab2(hs_new_p, Wp, bp.reshape(1, 128), W2, hs2f)

  p1a, p1b_, q2, ov = _sc_agg(tab1p, tab2p, pool_tab, s1m, d1m, s1e, d1e,
                              si2_3, di2_3, sip, dip, zeros128)

  fw_row = jnp.zeros((1, 128), f32).at[0, :2].set(fusion_weights[0])
  out = _tc_final(p1a, p1b_, tab1p, hd1f, q2, hd2f, ov,
                  b1.reshape(1, 128), b2.reshape(1, 128),
                  g1.reshape(1, 128), be1.reshape(1, 128),
                  g2.reshape(1, 128), be2.reshape(1, 128), Wm1[:128],
                  Wm1[128:], bm1.reshape(1, 256), Wm2, bm2.reshape(1, 128),
                  fw_row)
  return out
